# baseline (speedup 1.0000x reference)
import functools

import jax
import jax.numpy as jnp
from jax import lax
from jax.experimental import pallas as pl
from jax.experimental.pallas import tpu as pltpu
from jax.experimental.pallas import tpu_sc as plsc

_D = 110
_DP = 128
_L = 128


@functools.lru_cache(maxsize=None)
def _build_gather(N: int, J: int, V: int):
    info = plsc.get_sparse_core_info()
    NC, NS = info.num_cores, info.num_subcores
    NW = NC * NS
    n_it = N // _L
    it_per_w = n_it // NW
    n_super = J * it_per_w
    mesh = plsc.VectorSubcoreMesh(core_axis_name="c", subcore_axis_name="s")

    @functools.partial(
        pl.kernel,
        out_type=jax.ShapeDtypeStruct((J, _D, N), jnp.float32),
        mesh=mesh,
        scratch_types=[
            pltpu.VMEM((it_per_w, J, _L), jnp.int32),
            pltpu.VMEM((2, _L, _DP), jnp.float32),
            pltpu.VMEM((2, _D, _L), jnp.float32),
            pltpu.SemaphoreType.DMA,
            pltpu.SemaphoreType.DMA,
            pltpu.SemaphoreType.DMA,
            pltpu.SemaphoreType.DMA,
        ],
        compiler_params=pltpu.CompilerParams(needs_layout_passes=False),
    )
    def gather(steps_hbm, table_hbm, out_hbm, idx_v, pad_v, cmp_v, g0, g1, w0, w1):
        sem_g = (g0, g1)
        sem_w = (w0, w1)
        wid = lax.axis_index("s") * NC + lax.axis_index("c")
        it0 = wid * it_per_w
        pltpu.sync_copy(steps_hbm.at[pl.ds(it0, it_per_w)], idx_v)
        lane = jax.lax.iota(jnp.int32, 16)
        ivecs = [lane + g * 16 for g in range(_L // 16)]

        def gather_desc(t, b):
            j = t // it_per_w
            itl = lax.rem(t, it_per_w)
            return pltpu.make_async_copy(
                table_hbm.at[idx_v.at[itl, j]], pad_v.at[b], sem_g[b]
            )

        def write_desc(t, b):
            j = t // it_per_w
            itl = lax.rem(t, it_per_w)
            return pltpu.make_async_copy(
                cmp_v.at[b],
                out_hbm.at[j].at[:, pl.ds((it0 + itl) * _L, _L)],
                sem_w[b],
            )

        def transpose(b):
            src = pad_v.at[b]
            dst = cmp_v.at[b]

            def col_body(u2, kvec16):
                k0 = u2 * 2
                for d in (0, 1):
                    drow = dst.at[k0 + d]
                    kv = kvec16 + d
                    for g in range(_L // 16):
                        v = plsc.load_gather(src, [ivecs[g], kv])
                        drow[pl.ds(g * 16, 16)] = v
                return kvec16 + 2

            lax.fori_loop(0, _D // 2, col_body, lane * 0)

        for b in (0, 1):
            gather_desc(b, b).start()

        def pair_body(u, carry):
            for b in (0, 1):
                t = 2 * u + b
                gather_desc(t, b).wait()

                @pl.when(t >= 2)
                def _():
                    write_desc(t - 2, b).wait()

                transpose(b)
                write_desc(t, b).start()

                @pl.when(t + 2 < n_super)
                def _():
                    gather_desc(t + 2, b).start()
            return carry

        lax.fori_loop(0, n_super // 2, pair_body, 0)
        for b in (0, 1):
            write_desc(n_super - 2 + b, b).wait()

    return gather


def kernel(steps, embedding):
    N, J = steps.shape
    V, D = embedding.shape
    steps3 = steps.reshape(N // _L, _L, J).transpose(0, 2, 1)
    emb_p = jnp.pad(embedding, ((0, 0), (0, _DP - D)))
    out_t = _build_gather(N, J, V)(steps3, emb_p)
    return jnp.transpose(out_t, (2, 0, 1))


# speedup vs baseline: 2.6530x; 2.6530x over previous
import functools

import jax
import jax.numpy as jnp
from jax import lax
from jax.experimental import pallas as pl
from jax.experimental.pallas import tpu as pltpu
from jax.experimental.pallas import tpu_sc as plsc

_D = 110
_VP = 384
_L = 128


@functools.lru_cache(maxsize=None)
def _build_gather(N: int, J: int, V: int):
    info = plsc.get_sparse_core_info()
    NC, NS = info.num_cores, info.num_subcores
    NW = NC * NS
    n_it = N // _L
    it_per_w = n_it // NW
    n_super = J * it_per_w
    mesh = plsc.VectorSubcoreMesh(core_axis_name="c", subcore_axis_name="s")

    @functools.partial(
        pl.kernel,
        out_type=jax.ShapeDtypeStruct((J, _D, N), jnp.float32),
        mesh=mesh,
        scratch_types=[
            pltpu.VMEM((it_per_w, J, _L), jnp.int32),
            pltpu.VMEM((_D * _VP,), jnp.float32),
            pltpu.VMEM((2, _D, _L), jnp.float32),
            pltpu.SemaphoreType.DMA,
            pltpu.SemaphoreType.DMA,
        ],
        compiler_params=pltpu.CompilerParams(needs_layout_passes=False),
    )
    def gather(steps_hbm, table_hbm, out_hbm, idx_v, tab_v, cmp_v, w0, w1):
        sem_w = (w0, w1)
        wid = lax.axis_index("s") * NC + lax.axis_index("c")
        it0 = wid * it_per_w
        pltpu.sync_copy(table_hbm, tab_v)
        pltpu.sync_copy(steps_hbm.at[pl.ds(it0, it_per_w)], idx_v)
        lane = jax.lax.iota(jnp.int32, 16)
        kzero = lane * 0

        def write_desc(t, b):
            j = t // it_per_w
            itl = lax.rem(t, it_per_w)
            return pltpu.make_async_copy(
                cmp_v.at[b],
                out_hbm.at[j].at[:, pl.ds((it0 + itl) * _L, _L)],
                sem_w[b],
            )

        def transpose_gather(t, b):
            j = t // it_per_w
            itl = lax.rem(t, it_per_w)
            idx_row = idx_v.at[itl, j]
            ivs = [idx_row[pl.ds(g * 16, 16)] for g in range(_L // 16)]
            dst = cmp_v.at[b]

            def col_body(u2, kofs):
                k0 = u2 * 2
                for d in (0, 1):
                    drow = dst.at[k0 + d]
                    ko = kofs + d * _VP
                    for g in range(_L // 16):
                        v = plsc.load_gather(tab_v, [ko + ivs[g]])
                        drow[pl.ds(g * 16, 16)] = v
                return kofs + 2 * _VP

            lax.fori_loop(0, _D // 2, col_body, kzero)

        def pair_body(u, carry):
            for b in (0, 1):
                t = 2 * u + b

                @pl.when(t >= 2)
                def _():
                    write_desc(t - 2, b).wait()

                transpose_gather(t, b)
                write_desc(t, b).start()
            return carry

        lax.fori_loop(0, n_super // 2, pair_body, 0)
        for b in (0, 1):
            write_desc(n_super - 2 + b, b).wait()

    return gather


def kernel(steps, embedding):
    N, J = steps.shape
    V, D = embedding.shape
    steps3 = steps.reshape(N // _L, _L, J).transpose(0, 2, 1)
    tab_t = jnp.pad(embedding.T, ((0, 0), (0, _VP - V))).reshape(-1)
    out_t = _build_gather(N, J, V)(steps3, tab_t)
    return jnp.transpose(out_t, (2, 0, 1))
